# baseline (speedup 1.0000x reference)
import jax
import jax.numpy as jnp
from jax import lax
from jax.experimental import pallas as pl
from jax.experimental.pallas import tpu as pltpu
from jax.experimental.pallas import tpu_sc as plsc

_N = 10000
_E = 320000
_D1 = 128
_D2 = 48
_NC = 2
_NS = 16
_NW = _NC * _NS
_EPT = _E // _NW
_CH = 125
_NCH = _EPT // _CH
_DCH = 500
_DNCH = _EPT // _DCH
_SLAB = 8
_NP = 10240
_RPT = _NP // _NS
_ZB = 80
_ZC = _RPT // _ZB
_DEGW = 16

_mesh = plsc.VectorSubcoreMesh(
    core_axis_name="c", subcore_axis_name="s", num_cores=_NC, num_subcores=_NS
)


def _deg_body(src_hbm, dst_hbm, dego_hbm, degi_hbm, sidx, didx, buf, acco, acci,
              sem, sem2):
    c = lax.axis_index("c")
    s = lax.axis_index("s")
    wid = s * _NC + c

    pltpu.async_copy(src_hbm.at[wid], sidx, sem).wait()
    pltpu.async_copy(dst_hbm.at[wid], didx, sem).wait()

    @pl.loop(0, _DCH)
    def _zfill(r):
        buf[r] = jnp.zeros((_DEGW,), jnp.float32)

    @pl.loop(0, _ZC)
    def _zcopy(t):
        pltpu.sync_copy(buf.at[pl.ds(0, _ZB)],
                        acco.at[pl.ds(s * _RPT + t * _ZB, _ZB)])
        pltpu.sync_copy(buf.at[pl.ds(0, _ZB)],
                        acci.at[pl.ds(s * _RPT + t * _ZB, _ZB)])

    @pl.loop(0, _DCH)
    def _ofill(r):
        buf[r] = jnp.ones((_DEGW,), jnp.float32)

    plsc.subcore_barrier()

    @pl.loop(0, _DNCH)
    def _edges(j):
        a = pltpu.async_copy(buf, acco.at[sidx.at[j]], sem, add=True)
        b = pltpu.async_copy(buf, acci.at[didx.at[j]], sem2, add=True)
        a.wait()
        b.wait()

    plsc.subcore_barrier()
    pltpu.sync_copy(acco.at[pl.ds(s * _RPT, _RPT), pl.ds(0, 8)],
                    dego_hbm.at[c, pl.ds(s * _RPT, _RPT)])
    pltpu.sync_copy(acci.at[pl.ds(s * _RPT, _RPT), pl.ds(0, 8)],
                    degi_hbm.at[c, pl.ds(s * _RPT, _RPT)])


_deg = pl.kernel(
    _deg_body,
    out_type=[
        jax.ShapeDtypeStruct((_NC, _NP, 8), jnp.float32),
        jax.ShapeDtypeStruct((_NC, _NP, 8), jnp.float32),
    ],
    mesh=_mesh,
    compiler_params=pltpu.CompilerParams(use_tc_tiling_on_sc=False),
    scratch_types=[
        pltpu.VMEM((_DNCH, _DCH), jnp.int32),
        pltpu.VMEM((_DNCH, _DCH), jnp.int32),
        pltpu.VMEM((_DCH, _DEGW), jnp.float32),
        pltpu.VMEM_SHARED((_NP, _DEGW), jnp.float32),
        pltpu.VMEM_SHARED((_NP, _DEGW), jnp.float32),
        pltpu.SemaphoreType.DMA,
        pltpu.SemaphoreType.DMA,
    ],
)


def _spmm_body(d, ch, nch, slab, x_hbm, src_hbm, dst_hbm, out_hbm, s0, s1, s2,
               d0, d1, d2, rb0, rb1, acc, g0, g1, w0, w1, qsem):
    nslab = nch // slab
    c = lax.axis_index("c")
    s = lax.axis_index("s")
    wid = s * _NC + c

    @pl.loop(0, _ZB)
    def _zr(r):
        @pl.loop(0, d // 16)
        def _zc(k):
            rb0[r, pl.ds(k * 16, 16)] = jnp.zeros((16,), jnp.float32)

    @pl.loop(0, _ZC)
    def _zcopy(t):
        pltpu.sync_copy(rb0.at[pl.ds(0, _ZB)],
                        acc.at[pl.ds(s * _RPT + t * _ZB, _ZB)])

    plsc.subcore_barrier()

    sbufs, dbufs = (s0, s1, s2), (d0, d1, d2)
    gsems, wsems = (g0, g1), (w0, w1)
    rbs = (rb0, rb1)

    def slab_load(t):
        sb, db = sbufs[t % 3], dbufs[t % 3]
        a = pltpu.async_copy(src_hbm.at[wid, pl.ds(t * slab, slab)], sb, qsem)
        b = pltpu.async_copy(dst_hbm.at[wid, pl.ds(t * slab, slab)], db, qsem)
        return a, b

    def slab_wait(t):
        sb, db = sbufs[t % 3], dbufs[t % 3]
        pltpu.make_async_copy(src_hbm.at[wid, pl.ds(t * slab, slab)], sb,
                              qsem).wait()
        pltpu.make_async_copy(dst_hbm.at[wid, pl.ds(t * slab, slab)], db,
                              qsem).wait()

    def sidx(j):
        return sbufs[(j // slab) % 3].at[j % slab]

    def didx(j):
        return dbufs[(j // slab) % 3].at[j % slab]

    a, b = slab_load(0)
    a.wait()
    b.wait()
    slab_load(1)
    pltpu.async_copy(x_hbm.at[sidx(0)], rbs[0], gsems[0])

    for j in range(nch):
        p = j % 2
        pltpu.make_async_copy(x_hbm.at[sidx(j)], rbs[p], gsems[p]).wait()
        pltpu.async_copy(rbs[p], acc.at[didx(j)], wsems[p], add=True)
        jn = j + 1
        if jn < nch:
            if j >= 1:
                pltpu.make_async_copy(rbs[1 - p], acc.at[didx(j - 1)],
                                      wsems[1 - p]).wait()
            if jn % slab == 0:
                t = jn // slab
                slab_wait(t)
                if t + 1 < nslab:
                    slab_load(t + 1)
            pltpu.async_copy(x_hbm.at[sidx(jn)], rbs[1 - p], gsems[1 - p])

    pltpu.make_async_copy(rbs[0], acc.at[didx(nch - 2)], wsems[0]).wait()
    pltpu.make_async_copy(rbs[1], acc.at[didx(nch - 1)], wsems[1]).wait()

    plsc.subcore_barrier()
    pltpu.sync_copy(acc.at[pl.ds(s * _RPT, _RPT)],
                    out_hbm.at[c, pl.ds(s * _RPT, _RPT)])


def _make_spmm(d, ch, slab):
    params = None
    if d % 128 != 0:
        params = pltpu.CompilerParams(use_tc_tiling_on_sc=False)
    nch = _EPT // ch
    return pl.kernel(
        lambda *args: _spmm_body(d, ch, nch, slab, *args),
        out_type=jax.ShapeDtypeStruct((_NC, _NP, d), jnp.float32),
        mesh=_mesh,
        compiler_params=params,
        scratch_types=[
            pltpu.VMEM((slab, ch), jnp.int32),
            pltpu.VMEM((slab, ch), jnp.int32),
            pltpu.VMEM((slab, ch), jnp.int32),
            pltpu.VMEM((slab, ch), jnp.int32),
            pltpu.VMEM((slab, ch), jnp.int32),
            pltpu.VMEM((slab, ch), jnp.int32),
            pltpu.VMEM((ch, d), jnp.float32),
            pltpu.VMEM((ch, d), jnp.float32),
            pltpu.VMEM_SHARED((_NP, d), jnp.float32),
            pltpu.SemaphoreType.DMA,
            pltpu.SemaphoreType.DMA,
            pltpu.SemaphoreType.DMA,
            pltpu.SemaphoreType.DMA,
            pltpu.SemaphoreType.DMA,
        ],
    )


_CH1 = 125
_CH2 = 500
_spmm1 = _make_spmm(_D1, _CH1, 8)
_spmm2 = _make_spmm(_D2, _CH2, 4)


def _tc1a_body(x_ref, w1_ref, z_ref):
    z_ref[...] = jnp.dot(x_ref[...], w1_ref[...],
                         preferred_element_type=jnp.float32)


_tc1a = pl.pallas_call(
    _tc1a_body,
    out_shape=jax.ShapeDtypeStruct((_N, _D1), jnp.float32),
)


def _tc1b_body(z_ref, dego_ref, degi_ref, x1_ref, ni_ref, no_ref):
    dego = dego_ref[0, :_N, 0:1] + dego_ref[1, :_N, 0:1]
    degi = degi_ref[0, :_N, 0:1] + degi_ref[1, :_N, 0:1]
    no = lax.rsqrt(jnp.maximum(dego, 1.0))
    ni = lax.rsqrt(jnp.maximum(degi, 1.0))
    x1_ref[...] = z_ref[...] * no
    ni_ref[...] = ni
    no_ref[...] = no


_tc1b = pl.pallas_call(
    _tc1b_body,
    out_shape=[
        jax.ShapeDtypeStruct((_N, _D1), jnp.float32),
        jax.ShapeDtypeStruct((_N, 1), jnp.float32),
        jax.ShapeDtypeStruct((_N, 1), jnp.float32),
    ],
)


def _tc2_body(agg_ref, ni_ref, b1_ref, w2_ref, no_ref, m2_ref):
    h = (agg_ref[0, :_N] + agg_ref[1, :_N]) * ni_ref[...] + b1_ref[...]
    h = jnp.maximum(h, 0.0)
    m2 = jnp.dot(h, w2_ref[...], preferred_element_type=jnp.float32)
    m2_ref[:, :40] = m2 * no_ref[...]
    m2_ref[:, 40:] = jnp.zeros((_N, _D2 - 40), jnp.float32)


_tc2 = pl.pallas_call(
    _tc2_body,
    out_shape=jax.ShapeDtypeStruct((_N, _D2), jnp.float32),
)


def _tc3_body(agg_ref, ni_ref, b2_ref, out_ref):
    out_ref[...] = (agg_ref[0, :_N, :40] + agg_ref[1, :_N, :40]) * ni_ref[...] \
        + b2_ref[...]


_tc3 = pl.pallas_call(
    _tc3_body,
    out_shape=jax.ShapeDtypeStruct((_N, 40), jnp.float32),
)


def kernel(x, edge_index, W1, b1, W2, b2):
    src = edge_index[0].reshape(_NW, _NCH, _CH)
    dst = edge_index[1].reshape(_NW, _NCH, _CH)

    src2 = edge_index[0].reshape(_NW, _EPT // _CH2, _CH2)
    dst2 = edge_index[1].reshape(_NW, _EPT // _CH2, _CH2)

    dego, degi = _deg(src2, dst2)
    z = _tc1a(x, W1)
    x1, ni, no = _tc1b(z, dego, degi)
    agg1 = _spmm1(x1, src, dst)
    m2 = _tc2(agg1, ni, b1.reshape(1, -1), W2, no)
    agg2 = _spmm2(m2, src2, dst2)
    return _tc3(agg2, ni, b2.reshape(1, -1))


# speedup vs baseline: 1.0425x; 1.0425x over previous
import jax
import jax.numpy as jnp
from jax import lax
from jax.experimental import pallas as pl
from jax.experimental.pallas import tpu as pltpu
from jax.experimental.pallas import tpu_sc as plsc

_N = 10000
_E = 320000
_D1 = 128
_D2 = 48
_NC = 2
_NS = 16
_NW = _NC * _NS
_EPT = _E // _NW
_CH = 125
_NCH = _EPT // _CH
_DCH = 500
_DNCH = _EPT // _DCH
_SLAB = 8
_NP = 10240
_RPT = _NP // _NS
_ZB = 80
_ZC = _RPT // _ZB
_DEGW = 16

_mesh = plsc.VectorSubcoreMesh(
    core_axis_name="c", subcore_axis_name="s", num_cores=_NC, num_subcores=_NS
)


def _deg_body(src_hbm, dst_hbm, dego_hbm, degi_hbm, sidx, didx, buf, acco, acci,
              sem, sem2):
    c = lax.axis_index("c")
    s = lax.axis_index("s")
    wid = s * _NC + c

    pltpu.async_copy(src_hbm.at[wid], sidx, sem).wait()
    pltpu.async_copy(dst_hbm.at[wid], didx, sem).wait()

    @pl.loop(0, _DCH)
    def _zfill(r):
        buf[r] = jnp.zeros((_DEGW,), jnp.float32)

    @pl.loop(0, _ZC)
    def _zcopy(t):
        pltpu.sync_copy(buf.at[pl.ds(0, _ZB)],
                        acco.at[pl.ds(s * _RPT + t * _ZB, _ZB)])
        pltpu.sync_copy(buf.at[pl.ds(0, _ZB)],
                        acci.at[pl.ds(s * _RPT + t * _ZB, _ZB)])

    @pl.loop(0, _DCH)
    def _ofill(r):
        buf[r] = jnp.ones((_DEGW,), jnp.float32)

    plsc.subcore_barrier()

    @pl.loop(0, _DNCH)
    def _edges(j):
        a = pltpu.async_copy(buf, acco.at[sidx.at[j]], sem, add=True)
        b = pltpu.async_copy(buf, acci.at[didx.at[j]], sem2, add=True)
        a.wait()
        b.wait()

    plsc.subcore_barrier()
    pltpu.sync_copy(acco.at[pl.ds(s * _RPT, _RPT)],
                    dego_hbm.at[c, pl.ds(s * _RPT, _RPT)])
    pltpu.sync_copy(acci.at[pl.ds(s * _RPT, _RPT)],
                    degi_hbm.at[c, pl.ds(s * _RPT, _RPT)])


_deg = pl.kernel(
    _deg_body,
    out_type=[
        jax.ShapeDtypeStruct((_NC, _NP, _DEGW), jnp.float32),
        jax.ShapeDtypeStruct((_NC, _NP, _DEGW), jnp.float32),
    ],
    mesh=_mesh,
    compiler_params=pltpu.CompilerParams(use_tc_tiling_on_sc=False),
    scratch_types=[
        pltpu.VMEM((_DNCH, _DCH), jnp.int32),
        pltpu.VMEM((_DNCH, _DCH), jnp.int32),
        pltpu.VMEM((_DCH, _DEGW), jnp.float32),
        pltpu.VMEM_SHARED((_NP, _DEGW), jnp.float32),
        pltpu.VMEM_SHARED((_NP, _DEGW), jnp.float32),
        pltpu.SemaphoreType.DMA,
        pltpu.SemaphoreType.DMA,
    ],
)


def _spmm_body(d, ch, nch, slab, x_hbm, src_hbm, dst_hbm, out_hbm, s0, s1, s2,
               d0, d1, d2, rb0, rb1, acc, g0, g1, w0, w1, qsem):
    nslab = nch // slab
    c = lax.axis_index("c")
    s = lax.axis_index("s")
    wid = s * _NC + c

    @pl.loop(0, _ZB)
    def _zr(r):
        @pl.loop(0, d // 16)
        def _zc(k):
            rb0[r, pl.ds(k * 16, 16)] = jnp.zeros((16,), jnp.float32)

    @pl.loop(0, _ZC)
    def _zcopy(t):
        pltpu.sync_copy(rb0.at[pl.ds(0, _ZB)],
                        acc.at[pl.ds(s * _RPT + t * _ZB, _ZB)])

    plsc.subcore_barrier()

    sbufs, dbufs = (s0, s1, s2), (d0, d1, d2)
    gsems, wsems = (g0, g1), (w0, w1)
    rbs = (rb0, rb1)

    def slab_load(t):
        sb, db = sbufs[t % 3], dbufs[t % 3]
        a = pltpu.async_copy(src_hbm.at[wid, pl.ds(t * slab, slab)], sb, qsem)
        b = pltpu.async_copy(dst_hbm.at[wid, pl.ds(t * slab, slab)], db, qsem)
        return a, b

    def slab_wait(t):
        sb, db = sbufs[t % 3], dbufs[t % 3]
        pltpu.make_async_copy(src_hbm.at[wid, pl.ds(t * slab, slab)], sb,
                              qsem).wait()
        pltpu.make_async_copy(dst_hbm.at[wid, pl.ds(t * slab, slab)], db,
                              qsem).wait()

    def sidx(j):
        return sbufs[(j // slab) % 3].at[j % slab]

    def didx(j):
        return dbufs[(j // slab) % 3].at[j % slab]

    a, b = slab_load(0)
    a.wait()
    b.wait()
    slab_load(1)
    pltpu.async_copy(x_hbm.at[sidx(0)], rbs[0], gsems[0])

    for j in range(nch):
        p = j % 2
        pltpu.make_async_copy(x_hbm.at[sidx(j)], rbs[p], gsems[p]).wait()
        pltpu.async_copy(rbs[p], acc.at[didx(j)], wsems[p], add=True)
        jn = j + 1
        if jn < nch:
            if j >= 1:
                pltpu.make_async_copy(rbs[1 - p], acc.at[didx(j - 1)],
                                      wsems[1 - p]).wait()
            if jn % slab == 0:
                t = jn // slab
                slab_wait(t)
                if t + 1 < nslab:
                    slab_load(t + 1)
            pltpu.async_copy(x_hbm.at[sidx(jn)], rbs[1 - p], gsems[1 - p])

    pltpu.make_async_copy(rbs[0], acc.at[didx(nch - 2)], wsems[0]).wait()
    pltpu.make_async_copy(rbs[1], acc.at[didx(nch - 1)], wsems[1]).wait()

    plsc.subcore_barrier()
    pltpu.sync_copy(acc.at[pl.ds(s * _RPT, _RPT)],
                    out_hbm.at[c, pl.ds(s * _RPT, _RPT)])


def _make_spmm(d, ch, slab):
    params = None
    if d % 128 != 0:
        params = pltpu.CompilerParams(use_tc_tiling_on_sc=False)
    nch = _EPT // ch
    return pl.kernel(
        lambda *args: _spmm_body(d, ch, nch, slab, *args),
        out_type=jax.ShapeDtypeStruct((_NC, _NP, d), jnp.float32),
        mesh=_mesh,
        compiler_params=params,
        scratch_types=[
            pltpu.VMEM((slab, ch), jnp.int32),
            pltpu.VMEM((slab, ch), jnp.int32),
            pltpu.VMEM((slab, ch), jnp.int32),
            pltpu.VMEM((slab, ch), jnp.int32),
            pltpu.VMEM((slab, ch), jnp.int32),
            pltpu.VMEM((slab, ch), jnp.int32),
            pltpu.VMEM((ch, d), jnp.float32),
            pltpu.VMEM((ch, d), jnp.float32),
            pltpu.VMEM_SHARED((_NP, d), jnp.float32),
            pltpu.SemaphoreType.DMA,
            pltpu.SemaphoreType.DMA,
            pltpu.SemaphoreType.DMA,
            pltpu.SemaphoreType.DMA,
            pltpu.SemaphoreType.DMA,
        ],
    )


_CH1 = 125
_CH2 = 500
_spmm1 = _make_spmm(_D1, _CH1, 8)
_spmm2 = _make_spmm(_D2, _CH2, 4)


def _tc1a_body(x_ref, w1_ref, z_ref):
    z_ref[...] = jnp.dot(x_ref[...], w1_ref[...],
                         preferred_element_type=jnp.float32)


_tc1a = pl.pallas_call(
    _tc1a_body,
    out_shape=jax.ShapeDtypeStruct((_N, _D1), jnp.float32),
)


def _tc1b_body(z_ref, dego_ref, degi_ref, x1_ref, ni_ref, no_ref):
    dego = dego_ref[0, :_N, 0:1] + dego_ref[1, :_N, 0:1]
    degi = degi_ref[0, :_N, 0:1] + degi_ref[1, :_N, 0:1]
    no = lax.rsqrt(jnp.maximum(dego, 1.0))
    ni = lax.rsqrt(jnp.maximum(degi, 1.0))
    x1_ref[...] = z_ref[...] * no
    ni_ref[...] = ni
    no_ref[...] = no


_tc1b = pl.pallas_call(
    _tc1b_body,
    out_shape=[
        jax.ShapeDtypeStruct((_N, _D1), jnp.float32),
        jax.ShapeDtypeStruct((_N, 1), jnp.float32),
        jax.ShapeDtypeStruct((_N, 1), jnp.float32),
    ],
)


def _tc2_body(agg_ref, ni_ref, b1_ref, w2_ref, no_ref, m2_ref):
    h = (agg_ref[0, :_N] + agg_ref[1, :_N]) * ni_ref[...] + b1_ref[...]
    h = jnp.maximum(h, 0.0)
    m2 = jnp.dot(h, w2_ref[...], preferred_element_type=jnp.float32)
    m2_ref[:, :40] = m2 * no_ref[...]
    m2_ref[:, 40:] = jnp.zeros((_N, _D2 - 40), jnp.float32)


_tc2 = pl.pallas_call(
    _tc2_body,
    out_shape=jax.ShapeDtypeStruct((_N, _D2), jnp.float32),
)


def _tc3_body(agg_ref, ni_ref, b2_ref, out_ref):
    out_ref[...] = (agg_ref[0, :_N, :40] + agg_ref[1, :_N, :40]) * ni_ref[...] \
        + b2_ref[...]


_tc3 = pl.pallas_call(
    _tc3_body,
    out_shape=jax.ShapeDtypeStruct((_N, 40), jnp.float32),
)


def kernel(x, edge_index, W1, b1, W2, b2):
    src = edge_index[0].reshape(_NW, _NCH, _CH)
    dst = edge_index[1].reshape(_NW, _NCH, _CH)

    src2 = edge_index[0].reshape(_NW, _EPT // _CH2, _CH2)
    dst2 = edge_index[1].reshape(_NW, _EPT // _CH2, _CH2)

    dego, degi = _deg(src2, dst2)
    z = _tc1a(x, W1)
    x1, ni, no = _tc1b(z, dego, degi)
    agg1 = _spmm1(x1, src, dst)
    m2 = _tc2(agg1, ni, b1.reshape(1, -1), W2, no)
    agg2 = _spmm2(m2, src2, dst2)
    return _tc3(agg2, ni, b2.reshape(1, -1))
